# Initial kernel scaffold; baseline (speedup 1.0000x reference)
import jax
import jax.numpy as jnp
from jax.experimental import pallas as pl


def kernel(x, edge_index, W1, b1, W2, b2):
    raise NotImplementedError("write your pallas kernel here")


# speedup vs baseline: 26.5782x; 26.5782x over previous
import functools

import jax
import jax.numpy as jnp
from jax import lax
from jax.experimental import pallas as pl
from jax.experimental.pallas import tpu as pltpu
from jax.experimental.pallas import tpu_sc as plsc

N = 10000
NP = 10240
E = 320000
DF = 128
DH = 64
DO = 5
WPAD = 16

NC = 2
NS = 16
NW = NC * NS
EW = E // NW
CH = 80
NCHUNK = EW // CH
RPT = NP // NS

_MESH = plsc.VectorSubcoreMesh(
    core_axis_name="c", subcore_axis_name="s", num_cores=NC, num_subcores=NS)
_SC_PARAMS = pltpu.CompilerParams(use_tc_tiling_on_sc=False)


def _sc_edge_agg(width):
  @functools.partial(
      pl.kernel,
      out_type=jax.ShapeDtypeStruct((NC, NP, width), jnp.float32),
      mesh=_MESH,
      compiler_params=_SC_PARAMS,
      scratch_types=[
          pltpu.VMEM_SHARED((NP, width), jnp.float32),
          pltpu.VMEM((NCHUNK, CH), jnp.int32),
          pltpu.VMEM((NCHUNK, CH), jnp.int32),
          pltpu.VMEM((CH, width), jnp.float32),
          pltpu.SemaphoreType.DMA,
      ],
  )
  def body(table_hbm, src_hbm, dst_hbm, out_hbm, acc, srcs, dsts, rows, sem):
    c = lax.axis_index("c")
    s = lax.axis_index("s")
    wid = c * NS + s
    pltpu.sync_copy(table_hbm.at[pl.ds(s * RPT, RPT)],
                    acc.at[pl.ds(s * RPT, RPT)])
    pltpu.sync_copy(src_hbm.at[wid], srcs)
    pltpu.sync_copy(dst_hbm.at[wid], dsts)
    plsc.subcore_barrier()

    def step(j, _):
      pltpu.async_copy(table_hbm.at[srcs.at[j]], rows, sem).wait()
      pltpu.sync_copy(rows, acc.at[dsts.at[j]], add=True)
      return _

    lax.fori_loop(0, NCHUNK, step, None)
    plsc.subcore_barrier()
    pltpu.sync_copy(acc.at[pl.ds(s * RPT, RPT)],
                    out_hbm.at[c, pl.ds(s * RPT, RPT)])

  return body


@functools.partial(
    pl.kernel,
    out_type=jax.ShapeDtypeStruct((NC, NP, WPAD), jnp.float32),
    mesh=_MESH,
    compiler_params=_SC_PARAMS,
    scratch_types=[
        pltpu.VMEM_SHARED((NP, WPAD), jnp.float32),
        pltpu.VMEM((NCHUNK, CH), jnp.int32),
        pltpu.VMEM((CH, WPAD), jnp.float32),
    ],
)
def _sc_degree(ones_hbm, dst_hbm, out_hbm, acc, dsts, rows):
  c = lax.axis_index("c")
  s = lax.axis_index("s")
  wid = c * NS + s
  pltpu.sync_copy(ones_hbm.at[pl.ds(s * RPT, RPT)],
                  acc.at[pl.ds(s * RPT, RPT)])
  pltpu.sync_copy(dst_hbm.at[wid], dsts)
  pltpu.sync_copy(ones_hbm.at[pl.ds(0, CH)], rows)
  plsc.subcore_barrier()

  def step(j, _):
    pltpu.sync_copy(rows, acc.at[dsts.at[j]], add=True)
    return _

  lax.fori_loop(0, NCHUNK, step, None)
  plsc.subcore_barrier()
  pltpu.sync_copy(acc.at[pl.ds(s * RPT, RPT)],
                  out_hbm.at[c, pl.ds(s * RPT, RPT)])


def _tc1_body(x_ref, w1_ref, degp_ref, hs_ref, dinv_ref):
  deg = degp_ref[0, :, :1] + degp_ref[1, :, :1] - 1.0
  dinv = lax.rsqrt(deg)
  h = jnp.dot(x_ref[...], w1_ref[...], preferred_element_type=jnp.float32)
  hs_ref[...] = h * dinv
  dinv_ref[...] = dinv


def _tc2_body(p_ref, hs_ref, dinv_ref, b1_ref, w2p_ref, zs_ref):
  dinv = dinv_ref[...]
  a = dinv * (p_ref[0] + p_ref[1] - hs_ref[...]) + b1_ref[...]
  r = jnp.maximum(a, 0.0)
  z = jnp.dot(r, w2p_ref[...], preferred_element_type=jnp.float32)
  zs_ref[...] = z * dinv


def _tc3_body(q_ref, zs_ref, dinv_ref, b2p_ref, o_ref):
  a = dinv_ref[...] * (q_ref[0] + q_ref[1] - zs_ref[...]) + b2p_ref[...]
  col = lax.broadcasted_iota(jnp.int32, a.shape, 1)
  l = jnp.where(col < DO, a, -jnp.inf)
  m = jnp.max(l, axis=1, keepdims=True)
  ssum = jnp.sum(jnp.where(col < DO, jnp.exp(l - m), 0.0),
                 axis=1, keepdims=True)
  o_ref[...] = (l - m - jnp.log(ssum))[:, :DO]


_agg64 = _sc_edge_agg(DH)
_agg16 = _sc_edge_agg(WPAD)

_tc1 = pl.pallas_call(
    _tc1_body,
    out_shape=[jax.ShapeDtypeStruct((NP, DH), jnp.float32),
               jax.ShapeDtypeStruct((NP, 1), jnp.float32)])
_tc2 = pl.pallas_call(
    _tc2_body,
    out_shape=jax.ShapeDtypeStruct((NP, WPAD), jnp.float32))
_tc3 = pl.pallas_call(
    _tc3_body,
    out_shape=jax.ShapeDtypeStruct((NP, DO), jnp.float32))


@jax.jit
def kernel(x, edge_index, W1, b1, W2, b2):
  src = edge_index[0].astype(jnp.int32).reshape(NW, NCHUNK, CH)
  dst = edge_index[1].astype(jnp.int32).reshape(NW, NCHUNK, CH)
  xp = jnp.pad(x, ((0, NP - N), (0, 0)))
  ones16 = jnp.ones((NP, WPAD), jnp.float32)
  W2p = jnp.pad(W2, ((0, 0), (0, WPAD - DO)))
  b1r = b1.reshape(1, DH)
  b2p = jnp.pad(b2, (0, WPAD - DO)).reshape(1, WPAD)

  degp = _sc_degree(ones16, dst)
  hs, dinv = _tc1(xp, W1, degp)
  p = _agg64(hs, src, dst)
  zs = _tc2(p, hs, dinv, b1r, W2p)
  q = _agg16(zs, src, dst)
  return _tc3(q, zs, dinv, b2p)[:N]
